# baseline (speedup 1.0000x reference)
import jax
import jax.numpy as jnp
from jax import lax
from jax.experimental import pallas as pl
from jax.experimental.pallas import tpu as pltpu
from jax.experimental.pallas import tpu_sc as plsc

BATCH = 64
SEQ = 1025
PATCH = 1024
DIM = 768
KEEP = 512
OUT = KEEP + 1

NC, NS, L = 2, 16, 16
HB = BATCH // NC
RPT = HB // NS
SPT = KEEP // NS
CH = HB
NBUF = 4
NCHV = PATCH // L
PMS = 1024


def _body(x_hbm, noise_hbm, out_hbm, pm_hbm, noise_v, list_v, stage_v, blk_v,
          g0, g1, g2, g3, b0, b1, b2, b3,
          gs0, gs1, gs2, gs3, ss0, ss1, ss2, ss3):
    sc = lax.axis_index("c")
    tid = lax.axis_index("s")
    boff = sc * HB

    zeros = jnp.zeros((L,), jnp.int32)
    ones = jnp.full((L,), 1, jnp.int32)
    kvec = jnp.full((L,), KEEP, jnp.int32)
    lanes = lax.iota(jnp.int32, L)

    bloc0 = tid * RPT
    blk8 = boff + (bloc0 // 8) * 8
    pltpu.sync_copy(noise_hbm.at[pl.ds(blk8, 8)], noise_v)
    roff = (boff + bloc0) - blk8

    for r in range(RPT):
        rvec = jnp.full((L,), roff + r, jnp.int32)

        def nbits(i):
            v = plsc.load_gather(noise_v, [rvec, i * L + lanes])
            return plsc.bitcast(v, jnp.int32)

        def count_le(t_vec):
            def cbody(i, acc):
                return acc + plsc.all_reduce_population_count(nbits(i) <= t_vec)

            return lax.fori_loop(0, NCHV, cbody, zeros, unroll=4)

        def sbody(_, lohi):
            lo, hi = lohi
            mid = lo + lax.shift_right_logical(hi - lo, 1)
            pred = count_le(mid) >= kvec
            return jnp.where(pred, lo, mid + 1), jnp.where(pred, mid, hi)

        _, tstar = lax.fori_loop(
            0, 31, sbody, (zeros, jnp.full((L,), 0x7FFFFFFF, jnp.int32))
        )

        def cbody_lt(i, acc):
            return acc + plsc.all_reduce_population_count(nbits(i) < tstar)

        m = lax.fori_loop(0, NCHV, cbody_lt, zeros, unroll=4)
        need_eq = kvec - m

        plsc.store_scatter(list_v, [zeros], zeros, mask=lanes == zeros)

        def compact(i, carry):
            kept, eqs = carry
            bits = nbits(i)
            is_lt = bits < tstar
            is_eq = bits == tstar
            eq_i = jnp.where(is_eq, ones, zeros)
            eq_rank = plsc.cumsum(eq_i) - eq_i + eqs
            keep = is_lt | (is_eq & (eq_rank < need_eq))
            k_i = jnp.where(keep, ones, zeros)
            pos = plsc.cumsum(k_i) - k_i + kept
            lidx = i * L + lanes
            plsc.store_scatter(list_v, [pos + 1], lidx, mask=keep)
            return (
                kept + plsc.all_reduce_population_count(keep),
                eqs + plsc.all_reduce_population_count(is_eq),
            )

        lax.fori_loop(0, NCHV, compact, (zeros, zeros))

        pltpu.sync_copy(
            list_v, pm_hbm.at[pl.ds((boff + bloc0 + r) * PMS, OUT + 7)]
        )

    plsc.subcore_barrier()

    t0 = tid * SPT
    for st in range(HB // 8):
        pltpu.sync_copy(
            pm_hbm.at[pl.ds((boff + st * 8) * PMS, 8 * PMS)], stage_v
        )
        for bl in range(8):
            base = jnp.full((L,), bl * PMS + t0, jnp.int32)
            for j in range(SPT // L):
                s = plsc.load_gather(stage_v, [base + (j * L + lanes)])
                plsc.store_scatter(
                    blk_v, [j * L + lanes, jnp.full((L,), st * 8 + bl,
                                                    jnp.int32)], s
                )

    gidxs = (g0, g1, g2, g3)
    bufs = (b0, b1, b2, b3)
    gsems = (gs0, gs1, gs2, gs3)
    ssems = (ss0, ss1, ss2, ss3)

    def build_gidx(i):
        gidx = gidxs[i % NBUF]
        for j in range(CH // L):
            bvec = jnp.full((L,), j * L, jnp.int32) + lanes
            s = plsc.load_gather(blk_v, [jnp.full((L,), i, jnp.int32), bvec])
            gidx[pl.ds(j * L, L)] = s * BATCH + (boff + bvec)

    def orow(i):
        return (t0 + i) * BATCH + boff

    def g_issue(i):
        pltpu.async_copy(x_hbm.at[gidxs[i % NBUF]], bufs[i % NBUF],
                         gsems[i % NBUF])

    def g_wait(i):
        pltpu.make_async_copy(x_hbm.at[gidxs[i % NBUF]], bufs[i % NBUF],
                              gsems[i % NBUF]).wait()

    def s_issue(i):
        pltpu.async_copy(bufs[i % NBUF], out_hbm.at[pl.ds(orow(i), CH)],
                         ssems[i % NBUF])

    def s_wait(i):
        pltpu.make_async_copy(bufs[i % NBUF],
                              out_hbm.at[pl.ds(orow(i), CH)],
                              ssems[i % NBUF]).wait()

    for i in range(SPT + 2):
        if i < SPT:
            if i >= NBUF:
                s_wait(i - NBUF)
            build_gidx(i)
            g_issue(i)
        if i >= 2:
            g_wait(i - 2)
            s_issue(i - 2)
    for i in range(SPT - NBUF, SPT):
        s_wait(i)

    @pl.when(tid < 4)
    def _():
        pltpu.sync_copy(
            pm_hbm.at[pl.ds((boff + tid * 8) * PMS, 8 * PMS)], stage_v
        )
        msk = lanes < jnp.full((L,), 8, jnp.int32)
        flat = lanes * PMS + jnp.full((L,), KEEP, jnp.int32)
        s = plsc.load_gather(stage_v, [flat], mask=msk)
        g0[pl.ds(0, L)] = s * BATCH + (boff + tid * 8 + lanes)
        src = x_hbm.at[g0.at[pl.ds(0, 8)]]
        pltpu.async_copy(src, b0.at[pl.ds(0, 8)], gs0)
        pltpu.make_async_copy(src, b0.at[pl.ds(0, 8)], gs0).wait()
        dst = out_hbm.at[pl.ds(KEEP * BATCH + boff + tid * 8, 8)]
        pltpu.async_copy(b0.at[pl.ds(0, 8)], dst, ss0)
        pltpu.make_async_copy(b0.at[pl.ds(0, 8)], dst, ss0).wait()


_mesh = plsc.VectorSubcoreMesh(
    core_axis_name="c", subcore_axis_name="s", num_cores=NC, num_subcores=NS
)


@jax.jit
def _run(x, noise):
    f = pl.kernel(
        _body,
        out_type=(
            jax.ShapeDtypeStruct((OUT * BATCH, DIM), jnp.float32),
            jax.ShapeDtypeStruct((BATCH * PMS,), jnp.int32),
        ),
        mesh=_mesh,
        scratch_types=(
            [
                pltpu.VMEM((8, PATCH), jnp.float32),
                pltpu.VMEM((OUT + 7,), jnp.int32),
                pltpu.VMEM((8 * PMS,), jnp.int32),
                pltpu.VMEM((SPT, HB), jnp.int32),
            ]
            + [pltpu.VMEM((CH,), jnp.int32)] * NBUF
            + [pltpu.VMEM((CH, DIM), jnp.float32)] * NBUF
            + [pltpu.SemaphoreType.DMA] * (2 * NBUF)
        ),
        compiler_params=pltpu.CompilerParams(needs_layout_passes=False),
    )
    x2d = x.transpose(1, 0, 2).reshape(SEQ * BATCH, DIM)
    out2d, _ = f(x2d, noise)
    return out2d.reshape(OUT, BATCH, DIM).transpose(1, 0, 2)


def kernel(x, force_drop, noise):
    del force_drop
    return _run(x, noise)


# speedup vs baseline: 1.0245x; 1.0245x over previous
import jax
import jax.numpy as jnp
from jax import lax
from jax.experimental import pallas as pl
from jax.experimental.pallas import tpu as pltpu
from jax.experimental.pallas import tpu_sc as plsc

BATCH = 64
SEQ = 1025
PATCH = 1024
DIM = 768
KEEP = 512
OUT = KEEP + 1

NC, NS, L = 2, 16, 16
NW = NC * NS
RPT = BATCH // NW
SPT = KEEP // NW
NCHV = PATCH // L
PMS = 1024


def _sel_body(noise_hbm, pm_hbm, noise_v, list_v):
    wid = lax.axis_index("s") * NC + lax.axis_index("c")
    b0 = wid * RPT

    zeros = jnp.zeros((L,), jnp.int32)
    ones = jnp.full((L,), 1, jnp.int32)
    kvec = jnp.full((L,), KEEP, jnp.int32)
    lanes = lax.iota(jnp.int32, L)

    blk8 = (b0 // 8) * 8
    pltpu.sync_copy(noise_hbm.at[pl.ds(blk8, 8)], noise_v)

    for r in range(RPT):
        rvec = jnp.full((L,), b0 - blk8 + r, jnp.int32)

        def nbits(i):
            v = plsc.load_gather(noise_v, [rvec, i * L + lanes])
            return plsc.bitcast(v, jnp.int32)

        def count_le(t_vec):
            def cbody(i, acc):
                return acc + plsc.all_reduce_population_count(nbits(i) <= t_vec)

            return lax.fori_loop(0, NCHV, cbody, zeros, unroll=4)

        def sbody(_, lohi):
            lo, hi = lohi
            mid = lo + lax.shift_right_logical(hi - lo, 1)
            pred = count_le(mid) >= kvec
            return jnp.where(pred, lo, mid + 1), jnp.where(pred, mid, hi)

        _, tstar = lax.fori_loop(
            0, 31, sbody, (zeros, jnp.full((L,), 0x7FFFFFFF, jnp.int32))
        )

        def cbody_lt(i, acc):
            return acc + plsc.all_reduce_population_count(nbits(i) < tstar)

        m = lax.fori_loop(0, NCHV, cbody_lt, zeros, unroll=4)
        need_eq = kvec - m

        plsc.store_scatter(list_v, [jnp.full((L,), OUT, jnp.int32) + lanes],
                           zeros, mask=lanes < jnp.full((L,), 7, jnp.int32))
        plsc.store_scatter(list_v, [zeros], zeros, mask=lanes == zeros)

        def compact(i, carry):
            kept, eqs = carry
            bits = nbits(i)
            is_lt = bits < tstar
            is_eq = bits == tstar
            eq_i = jnp.where(is_eq, ones, zeros)
            eq_rank = plsc.cumsum(eq_i) - eq_i + eqs
            keep = is_lt | (is_eq & (eq_rank < need_eq))
            k_i = jnp.where(keep, ones, zeros)
            pos = plsc.cumsum(k_i) - k_i + kept
            lidx = i * L + lanes
            plsc.store_scatter(list_v, [pos + 1], lidx, mask=keep)
            return (
                kept + plsc.all_reduce_population_count(keep),
                eqs + plsc.all_reduce_population_count(is_eq),
            )

        lax.fori_loop(0, NCHV, compact, (zeros, zeros))

        pltpu.sync_copy(list_v, pm_hbm.at[pl.ds((b0 + r) * PMS, OUT + 7)])


CH = 32
NSUB = SPT * (BATCH // CH)
NBUF = 4


def _gat_body(x_hbm, pmT_hbm, out_hbm, blk_v,
              g0, g1, g2, g3, b0, b1, b2, b3,
              gs0, gs1, gs2, gs3, ss0, ss1, ss2, ss3):
    wid = lax.axis_index("s") * NC + lax.axis_index("c")
    t0 = wid * SPT
    lanes = lax.iota(jnp.int32, L)
    gidxs = (g0, g1, g2, g3)
    bufs = (b0, b1, b2, b3)
    gsems = (gs0, gs1, gs2, gs3)
    ssems = (ss0, ss1, ss2, ss3)

    pltpu.sync_copy(pmT_hbm.at[pl.ds(t0, SPT)], blk_v)

    def build_gidx(i, n_rows=CH):
        tt, hb = i // 2, (i % 2) * CH
        gidx = gidxs[i % NBUF]
        for j in range(n_rows // L):
            bvec = jnp.full((L,), hb + j * L, jnp.int32) + lanes
            s = plsc.load_gather(blk_v, [jnp.full((L,), tt, jnp.int32), bvec])
            gidx[pl.ds(j * L, L)] = s * BATCH + bvec

    def orow(i):
        return (t0 + i // 2) * BATCH + (i % 2) * CH

    def g_issue(i):
        pltpu.async_copy(x_hbm.at[gidxs[i % NBUF]], bufs[i % NBUF],
                         gsems[i % NBUF])

    def g_wait(i):
        pltpu.make_async_copy(x_hbm.at[gidxs[i % NBUF]], bufs[i % NBUF],
                              gsems[i % NBUF]).wait()

    def s_issue(i):
        pltpu.async_copy(bufs[i % NBUF], out_hbm.at[pl.ds(orow(i), CH)],
                         ssems[i % NBUF])

    def s_wait(i):
        pltpu.make_async_copy(bufs[i % NBUF],
                              out_hbm.at[pl.ds(orow(i), CH)],
                              ssems[i % NBUF]).wait()

    for i in range(NSUB + 2):
        if i < NSUB:
            if i >= NBUF:
                s_wait(i - NBUF)
            build_gidx(i)
            g_issue(i)
        if i >= 2:
            g_wait(i - 2)
            s_issue(i - 2)
    for i in range(NSUB - NBUF, NSUB):
        s_wait(i)

    @pl.when(wid < 8)
    def _():
        pltpu.sync_copy(pmT_hbm.at[pl.ds(KEEP, 8)], blk_v.at[pl.ds(0, 8)])
        bvec = wid * 8 + lanes
        s = plsc.load_gather(blk_v, [jnp.zeros((L,), jnp.int32), bvec],
                             mask=lanes < jnp.full((L,), 8, jnp.int32))
        g0[pl.ds(0, L)] = s * BATCH + bvec
        src = x_hbm.at[g0.at[pl.ds(0, 8)]]
        pltpu.async_copy(src, b0.at[pl.ds(0, 8)], gs0)
        pltpu.make_async_copy(src, b0.at[pl.ds(0, 8)], gs0).wait()
        dst = out_hbm.at[pl.ds(KEEP * BATCH + wid * 8, 8)]
        pltpu.async_copy(b0.at[pl.ds(0, 8)], dst, ss0)
        pltpu.make_async_copy(b0.at[pl.ds(0, 8)], dst, ss0).wait()


_mesh = plsc.VectorSubcoreMesh(
    core_axis_name="c", subcore_axis_name="s", num_cores=NC, num_subcores=NS
)


@jax.jit
def _run(x, noise):
    sel = pl.kernel(
        _sel_body,
        out_type=jax.ShapeDtypeStruct((BATCH * PMS,), jnp.int32),
        mesh=_mesh,
        scratch_types=[
            pltpu.VMEM((8, PATCH), jnp.float32),
            pltpu.VMEM((OUT + 7,), jnp.int32),
        ],
        compiler_params=pltpu.CompilerParams(needs_layout_passes=False),
    )
    pm = sel(noise)
    pmT = pm.reshape(BATCH, PMS).T

    gat = pl.kernel(
        _gat_body,
        out_type=jax.ShapeDtypeStruct((OUT * BATCH, DIM), jnp.float32),
        mesh=_mesh,
        scratch_types=(
            [pltpu.VMEM((L, BATCH), jnp.int32)]
            + [pltpu.VMEM((CH,), jnp.int32)] * NBUF
            + [pltpu.VMEM((CH, DIM), jnp.float32)] * NBUF
            + [pltpu.SemaphoreType.DMA] * (2 * NBUF)
        ),
        compiler_params=pltpu.CompilerParams(needs_layout_passes=False),
    )
    x2d = x.transpose(1, 0, 2).reshape(SEQ * BATCH, DIM)
    out2d = gat(x2d, pmT)
    return out2d.reshape(OUT, BATCH, DIM).transpose(1, 0, 2)


def kernel(x, force_drop, noise):
    del force_drop
    return _run(x, noise)


# speedup vs baseline: 1.0702x; 1.0446x over previous
import jax
import jax.numpy as jnp
from jax import lax
from jax.experimental import pallas as pl
from jax.experimental.pallas import tpu as pltpu
from jax.experimental.pallas import tpu_sc as plsc

BATCH = 64
SEQ = 1025
PATCH = 1024
DIM = 768
KEEP = 512
OUT = KEEP + 1

NC, NS, L = 2, 16, 16
NW = NC * NS
RPT = BATCH // NW
SPT = KEEP // NW
NCHV = PATCH // L
PMS = 1024


def _sel_body(noise_hbm, pm_hbm, noise_v, list_v):
    wid = lax.axis_index("s") * NC + lax.axis_index("c")
    b0 = wid * RPT

    zeros = jnp.zeros((L,), jnp.int32)
    ones = jnp.full((L,), 1, jnp.int32)
    kvec = jnp.full((L,), KEEP, jnp.int32)
    lanes = lax.iota(jnp.int32, L)

    for r in range(RPT):
        pltpu.sync_copy(noise_hbm.at[pl.ds((b0 + r) * PATCH, PATCH)], noise_v)

        def count_le(t_vec):
            def cbody(i, acc):
                bits = plsc.bitcast(noise_v[pl.ds(i * L, L)], jnp.int32)
                return acc + plsc.all_reduce_population_count(bits <= t_vec)

            return lax.fori_loop(0, NCHV, cbody, zeros, unroll=4)

        def sbody(_, lohi):
            lo, hi = lohi
            mid = lo + lax.shift_right_logical(hi - lo, 1)
            pred = count_le(mid) >= kvec
            return jnp.where(pred, lo, mid + 1), jnp.where(pred, mid, hi)

        _, tstar = lax.fori_loop(
            0, 31, sbody, (zeros, jnp.full((L,), 0x7FFFFFFF, jnp.int32))
        )

        def cbody_lt(i, acc):
            bits = plsc.bitcast(noise_v[pl.ds(i * L, L)], jnp.int32)
            return acc + plsc.all_reduce_population_count(bits < tstar)

        m = lax.fori_loop(0, NCHV, cbody_lt, zeros, unroll=4)
        need_eq = kvec - m

        plsc.store_scatter(list_v, [jnp.full((L,), OUT, jnp.int32) + lanes],
                           zeros, mask=lanes < jnp.full((L,), 7, jnp.int32))
        plsc.store_scatter(list_v, [zeros], zeros, mask=lanes == zeros)

        def compact(i, carry):
            kept, eqs = carry
            bits = plsc.bitcast(noise_v[pl.ds(i * L, L)], jnp.int32)
            is_lt = bits < tstar
            is_eq = bits == tstar
            eq_i = jnp.where(is_eq, ones, zeros)
            eq_rank = plsc.cumsum(eq_i) - eq_i + eqs
            keep = is_lt | (is_eq & (eq_rank < need_eq))
            k_i = jnp.where(keep, ones, zeros)
            pos = plsc.cumsum(k_i) - k_i + kept
            lidx = i * L + lanes
            plsc.store_scatter(list_v, [pos + 1], lidx, mask=keep)
            return (
                kept + plsc.all_reduce_population_count(keep),
                eqs + plsc.all_reduce_population_count(is_eq),
            )

        lax.fori_loop(0, NCHV, compact, (zeros, zeros))

        pltpu.sync_copy(list_v, pm_hbm.at[pl.ds((b0 + r) * PMS, OUT + 7)])


CH = 32
NSUB = SPT * (BATCH // CH)
NBUF = 4


def _gat_body(x_hbm, pmT_hbm, out_hbm, blk_v,
              g0, g1, g2, g3, b0, b1,
              gs0, gs1, gs2, gs3, ss0, ss1):
    wid = lax.axis_index("s") * NC + lax.axis_index("c")
    t0 = wid * SPT
    lanes = lax.iota(jnp.int32, L)
    gidxs = (g0, g1, g2, g3)
    bufs = (b0, b1)
    gsems = (gs0, gs1, gs2, gs3)
    ssems = (ss0, ss1)

    pltpu.sync_copy(pmT_hbm.at[pl.ds(t0, SPT)], blk_v)

    def build_gidx(k, h):
        gidx = gidxs[(2 * k + h) % 4]
        for j in range(CH // L):
            bvec = jnp.full((L,), h * CH + j * L, jnp.int32) + lanes
            s = plsc.load_gather(blk_v, [jnp.full((L,), k, jnp.int32), bvec])
            gidx[pl.ds(j * L, L)] = s * BATCH + bvec

    def g_issue(k, h):
        pltpu.async_copy(x_hbm.at[gidxs[(2 * k + h) % 4]],
                         bufs[k % 2].at[pl.ds(h * CH, CH)],
                         gsems[(2 * k + h) % 4])

    def g_wait(k, h):
        pltpu.make_async_copy(x_hbm.at[gidxs[(2 * k + h) % 4]],
                              bufs[k % 2].at[pl.ds(h * CH, CH)],
                              gsems[(2 * k + h) % 4]).wait()

    def s_issue(k):
        pltpu.async_copy(bufs[k % 2], out_hbm.at[pl.ds((t0 + k) * BATCH,
                                                       BATCH)], ssems[k % 2])

    def s_wait(k):
        pltpu.make_async_copy(bufs[k % 2],
                              out_hbm.at[pl.ds((t0 + k) * BATCH, BATCH)],
                              ssems[k % 2]).wait()

    for k in (0, 1):
        for h in (0, 1):
            build_gidx(k, h)
            g_issue(k, h)
    for k in range(SPT):
        g_wait(k, 0)
        g_wait(k, 1)
        s_issue(k)
        if k + 2 < SPT:
            s_wait(k)
            for h in (0, 1):
                build_gidx(k + 2, h)
                g_issue(k + 2, h)
    s_wait(SPT - 2)
    s_wait(SPT - 1)

    @pl.when(wid < 8)
    def _():
        pltpu.sync_copy(pmT_hbm.at[pl.ds(KEEP, 8)], blk_v.at[pl.ds(0, 8)])
        bvec = wid * 8 + lanes
        s = plsc.load_gather(blk_v, [jnp.zeros((L,), jnp.int32), bvec],
                             mask=lanes < jnp.full((L,), 8, jnp.int32))
        g0[pl.ds(0, L)] = s * BATCH + bvec
        src = x_hbm.at[g0.at[pl.ds(0, 8)]]
        pltpu.async_copy(src, b0.at[pl.ds(0, 8)], gs0)
        pltpu.make_async_copy(src, b0.at[pl.ds(0, 8)], gs0).wait()
        dst = out_hbm.at[pl.ds(KEEP * BATCH + wid * 8, 8)]
        pltpu.async_copy(b0.at[pl.ds(0, 8)], dst, ss0)
        pltpu.make_async_copy(b0.at[pl.ds(0, 8)], dst, ss0).wait()


_mesh = plsc.VectorSubcoreMesh(
    core_axis_name="c", subcore_axis_name="s", num_cores=NC, num_subcores=NS
)


@jax.jit
def _run(x, noise):
    sel = pl.kernel(
        _sel_body,
        out_type=jax.ShapeDtypeStruct((BATCH * PMS,), jnp.int32),
        mesh=_mesh,
        scratch_types=[
            pltpu.VMEM((PATCH,), jnp.float32),
            pltpu.VMEM((OUT + 7,), jnp.int32),
        ],
        compiler_params=pltpu.CompilerParams(needs_layout_passes=False),
    )
    pm = sel(noise.reshape(BATCH * PATCH))
    pmT = pm.reshape(BATCH, PMS).T

    gat = pl.kernel(
        _gat_body,
        out_type=jax.ShapeDtypeStruct((OUT * BATCH, DIM), jnp.float32),
        mesh=_mesh,
        scratch_types=(
            [pltpu.VMEM((L, BATCH), jnp.int32)]
            + [pltpu.VMEM((CH,), jnp.int32)] * 4
            + [pltpu.VMEM((BATCH, DIM), jnp.float32)] * 2
            + [pltpu.SemaphoreType.DMA] * 4
            + [pltpu.SemaphoreType.DMA] * 2
        ),
        compiler_params=pltpu.CompilerParams(needs_layout_passes=False),
    )
    x2d = x.transpose(1, 0, 2).reshape(SEQ * BATCH, DIM)
    out2d = gat(x2d, pmT)
    return out2d.reshape(OUT, BATCH, DIM).transpose(1, 0, 2)


def kernel(x, force_drop, noise):
    del force_drop
    return _run(x, noise)


# speedup vs baseline: 1.0746x; 1.0041x over previous
import jax
import jax.numpy as jnp
from jax import lax
from jax.experimental import pallas as pl
from jax.experimental.pallas import tpu as pltpu
from jax.experimental.pallas import tpu_sc as plsc

BATCH = 64
SEQ = 1025
PATCH = 1024
DIM = 768
KEEP = 512
OUT = KEEP + 1

NC, NS, L = 2, 16, 16
NW = NC * NS
RPT = BATCH // NW
SPT = KEEP // NW
NCHV = PATCH // L
PMS = 1024


def _sel_body(noise_hbm, pm_hbm, noise_v, list_v):
    wid = lax.axis_index("s") * NC + lax.axis_index("c")
    b0 = wid * RPT

    zeros = jnp.zeros((L,), jnp.int32)
    ones = jnp.full((L,), 1, jnp.int32)
    kvec = jnp.full((L,), KEEP, jnp.int32)
    lanes = lax.iota(jnp.int32, L)

    for r in range(RPT):
        pltpu.sync_copy(noise_hbm.at[pl.ds((b0 + r) * PATCH, PATCH)], noise_v)

        def count_le(t_vec):
            def cbody(i, acc):
                bits = plsc.bitcast(noise_v[pl.ds(i * L, L)], jnp.int32)
                return acc + plsc.all_reduce_population_count(bits <= t_vec)

            return lax.fori_loop(0, NCHV, cbody, zeros, unroll=4)

        def sbody(_, lohi):
            lo, hi = lohi
            mid = lo + lax.shift_right_logical(hi - lo, 1)
            pred = count_le(mid) >= kvec
            return jnp.where(pred, lo, mid + 1), jnp.where(pred, mid, hi)

        _, tstar = lax.fori_loop(
            0, 31, sbody, (zeros, jnp.full((L,), 0x7FFFFFFF, jnp.int32))
        )

        def cbody_lt(i, acc):
            bits = plsc.bitcast(noise_v[pl.ds(i * L, L)], jnp.int32)
            return acc + plsc.all_reduce_population_count(bits < tstar)

        m = lax.fori_loop(0, NCHV, cbody_lt, zeros, unroll=4)
        need_eq = kvec - m

        plsc.store_scatter(list_v, [jnp.full((L,), OUT, jnp.int32) + lanes],
                           zeros, mask=lanes < jnp.full((L,), 7, jnp.int32))
        plsc.store_scatter(list_v, [zeros], zeros, mask=lanes == zeros)

        def compact(i, carry):
            kept, eqs = carry
            bits = plsc.bitcast(noise_v[pl.ds(i * L, L)], jnp.int32)
            is_lt = bits < tstar
            is_eq = bits == tstar
            eq_i = jnp.where(is_eq, ones, zeros)
            eq_rank = plsc.cumsum(eq_i) - eq_i + eqs
            keep = is_lt | (is_eq & (eq_rank < need_eq))
            k_i = jnp.where(keep, ones, zeros)
            pos = plsc.cumsum(k_i) - k_i + kept
            lidx = i * L + lanes
            plsc.store_scatter(list_v, [pos + 1], lidx, mask=keep)
            return (
                kept + plsc.all_reduce_population_count(keep),
                eqs + plsc.all_reduce_population_count(is_eq),
            )

        lax.fori_loop(0, NCHV, compact, (zeros, zeros))

        pltpu.sync_copy(list_v, pm_hbm.at[pl.ds((b0 + r) * PMS, OUT + 7)])


CH = 32
NSUB = SPT * (BATCH // CH)
NBUF = 4


NQ = 4


def _gat_body(x_hbm, pmT_hbm, out_hbm, blk_v,
              g0, g1, g2, g3, g4, g5, g6, g7, b0, b1,
              gs0, gs1, gs2, gs3, gs4, gs5, gs6, gs7, ss0, ss1):
    wid = lax.axis_index("s") * NC + lax.axis_index("c")
    t0 = wid * SPT
    lanes = lax.iota(jnp.int32, L)
    gidxs = (g0, g1, g2, g3, g4, g5, g6, g7)
    bufs = (b0, b1)
    gsems = (gs0, gs1, gs2, gs3, gs4, gs5, gs6, gs7)
    ssems = (ss0, ss1)
    QR = BATCH // NQ

    pltpu.sync_copy(pmT_hbm.at[pl.ds(t0, SPT)], blk_v)

    def build_gidx(k, h):
        gidx = gidxs[(k % 2) * NQ + h]
        bvec = jnp.full((L,), h * QR, jnp.int32) + lanes
        s = plsc.load_gather(blk_v, [jnp.full((L,), k, jnp.int32), bvec])
        gidx[...] = s * BATCH + bvec

    def g_issue(k, h):
        pltpu.async_copy(x_hbm.at[gidxs[(k % 2) * NQ + h]],
                         bufs[k % 2].at[pl.ds(h * QR, QR)],
                         gsems[(k % 2) * NQ + h])

    def g_wait(k, h):
        pltpu.make_async_copy(x_hbm.at[gidxs[(k % 2) * NQ + h]],
                              bufs[k % 2].at[pl.ds(h * QR, QR)],
                              gsems[(k % 2) * NQ + h]).wait()

    def s_issue(k):
        pltpu.async_copy(bufs[k % 2], out_hbm.at[pl.ds((t0 + k) * BATCH,
                                                       BATCH)], ssems[k % 2])

    def s_wait(k):
        pltpu.make_async_copy(bufs[k % 2],
                              out_hbm.at[pl.ds((t0 + k) * BATCH, BATCH)],
                              ssems[k % 2]).wait()

    for k in (0, 1):
        for h in range(NQ):
            build_gidx(k, h)
            g_issue(k, h)
    for k in range(SPT):
        for h in range(NQ):
            g_wait(k, h)
        s_issue(k)
        if k + 2 < SPT:
            s_wait(k)
            for h in range(NQ):
                build_gidx(k + 2, h)
                g_issue(k + 2, h)
    s_wait(SPT - 2)
    s_wait(SPT - 1)

    @pl.when(wid < 8)
    def _():
        pltpu.sync_copy(pmT_hbm.at[pl.ds(KEEP, 8)], blk_v.at[pl.ds(0, 8)])
        bvec = wid * 8 + lanes
        s = plsc.load_gather(blk_v, [jnp.zeros((L,), jnp.int32), bvec],
                             mask=lanes < jnp.full((L,), 8, jnp.int32))
        g0[pl.ds(0, L)] = s * BATCH + bvec
        src = x_hbm.at[g0.at[pl.ds(0, 8)]]
        pltpu.async_copy(src, b0.at[pl.ds(0, 8)], gs0)
        pltpu.make_async_copy(src, b0.at[pl.ds(0, 8)], gs0).wait()
        dst = out_hbm.at[pl.ds(KEEP * BATCH + wid * 8, 8)]
        pltpu.async_copy(b0.at[pl.ds(0, 8)], dst, ss0)
        pltpu.make_async_copy(b0.at[pl.ds(0, 8)], dst, ss0).wait()


_mesh = plsc.VectorSubcoreMesh(
    core_axis_name="c", subcore_axis_name="s", num_cores=NC, num_subcores=NS
)


@jax.jit
def _run(x, noise):
    sel = pl.kernel(
        _sel_body,
        out_type=jax.ShapeDtypeStruct((BATCH * PMS,), jnp.int32),
        mesh=_mesh,
        scratch_types=[
            pltpu.VMEM((PATCH,), jnp.float32),
            pltpu.VMEM((OUT + 7,), jnp.int32),
        ],
        compiler_params=pltpu.CompilerParams(needs_layout_passes=False),
    )
    pm = sel(noise.reshape(BATCH * PATCH))
    pmT = pm.reshape(BATCH, PMS).T

    gat = pl.kernel(
        _gat_body,
        out_type=jax.ShapeDtypeStruct((OUT * BATCH, DIM), jnp.float32),
        mesh=_mesh,
        scratch_types=(
            [pltpu.VMEM((L, BATCH), jnp.int32)]
            + [pltpu.VMEM((L,), jnp.int32)] * 8
            + [pltpu.VMEM((BATCH, DIM), jnp.float32)] * 2
            + [pltpu.SemaphoreType.DMA] * 10
        ),
        compiler_params=pltpu.CompilerParams(needs_layout_passes=False),
    )
    x2d = x.transpose(1, 0, 2).reshape(SEQ * BATCH, DIM)
    out2d = gat(x2d, pmT)
    return out2d.reshape(OUT, BATCH, DIM).transpose(1, 0, 2)


def kernel(x, force_drop, noise):
    del force_drop
    return _run(x, noise)


# speedup vs baseline: 1.0908x; 1.0151x over previous
import jax
import jax.numpy as jnp
from jax import lax
from jax.experimental import pallas as pl
from jax.experimental.pallas import tpu as pltpu
from jax.experimental.pallas import tpu_sc as plsc

BATCH = 64
SEQ = 1025
PATCH = 1024
DIM = 768
KEEP = 512
OUT = KEEP + 1

NC, NS, L = 2, 16, 16
NW = NC * NS
RPT = BATCH // NW
SPT = KEEP // NW
NCHV = PATCH // L
PMS = 1024


def _sel_body(noise_hbm, pm_hbm, noise_v, list_v):
    wid = lax.axis_index("s") * NC + lax.axis_index("c")
    b0 = wid * RPT

    zeros = jnp.zeros((L,), jnp.int32)
    ones = jnp.full((L,), 1, jnp.int32)
    kvec = jnp.full((L,), KEEP, jnp.int32)
    lanes = lax.iota(jnp.int32, L)

    for r in range(RPT):
        pltpu.sync_copy(noise_hbm.at[pl.ds((b0 + r) * PATCH, PATCH)], noise_v)

        def count_le(t_vec):
            def cbody(i, acc):
                bits = plsc.bitcast(noise_v[pl.ds(i * L, L)], jnp.int32)
                return acc + plsc.all_reduce_population_count(bits <= t_vec)

            return lax.fori_loop(0, NCHV, cbody, zeros, unroll=8)

        def sbody(_, lohi):
            lo, hi = lohi
            mid = lo + lax.shift_right_logical(hi - lo, 1)
            pred = count_le(mid) >= kvec
            return jnp.where(pred, lo, mid + 1), jnp.where(pred, mid, hi)

        _, tstar = lax.fori_loop(
            0, 30, sbody, (zeros, jnp.full((L,), 0x3F800000, jnp.int32))
        )

        def cbody_lt(i, acc):
            bits = plsc.bitcast(noise_v[pl.ds(i * L, L)], jnp.int32)
            return acc + plsc.all_reduce_population_count(bits < tstar)

        m = lax.fori_loop(0, NCHV, cbody_lt, zeros, unroll=8)
        need_eq = kvec - m

        plsc.store_scatter(list_v, [jnp.full((L,), OUT, jnp.int32) + lanes],
                           zeros, mask=lanes < jnp.full((L,), 7, jnp.int32))
        plsc.store_scatter(list_v, [zeros], zeros, mask=lanes == zeros)

        def compact(i, carry):
            kept, eqs = carry
            bits = plsc.bitcast(noise_v[pl.ds(i * L, L)], jnp.int32)
            is_lt = bits < tstar
            is_eq = bits == tstar
            eq_i = jnp.where(is_eq, ones, zeros)
            eq_rank = plsc.cumsum(eq_i) - eq_i + eqs
            keep = is_lt | (is_eq & (eq_rank < need_eq))
            k_i = jnp.where(keep, ones, zeros)
            pos = plsc.cumsum(k_i) - k_i + kept
            lidx = i * L + lanes
            plsc.store_scatter(list_v, [pos + 1], lidx, mask=keep)
            return (
                kept + plsc.all_reduce_population_count(keep),
                eqs + plsc.all_reduce_population_count(is_eq),
            )

        lax.fori_loop(0, NCHV, compact, (zeros, zeros))

        pltpu.sync_copy(list_v, pm_hbm.at[pl.ds((b0 + r) * PMS, OUT + 7)])


CH = 32
NSUB = SPT * (BATCH // CH)
NBUF = 4


NQ = 4


def _gat_body(x_hbm, pmT_hbm, out_hbm, blk_v,
              g0, g1, g2, g3, g4, g5, g6, g7, b0, b1,
              gs0, gs1, gs2, gs3, gs4, gs5, gs6, gs7, ss0, ss1):
    wid = lax.axis_index("s") * NC + lax.axis_index("c")
    t0 = wid * SPT
    lanes = lax.iota(jnp.int32, L)
    gidxs = (g0, g1, g2, g3, g4, g5, g6, g7)
    bufs = (b0, b1)
    gsems = (gs0, gs1, gs2, gs3, gs4, gs5, gs6, gs7)
    ssems = (ss0, ss1)
    QR = BATCH // NQ

    pltpu.sync_copy(pmT_hbm.at[pl.ds(t0, SPT)], blk_v)

    def build_gidx(k, h):
        gidx = gidxs[(k % 2) * NQ + h]
        bvec = jnp.full((L,), h * QR, jnp.int32) + lanes
        s = plsc.load_gather(blk_v, [jnp.full((L,), k, jnp.int32), bvec])
        gidx[...] = s * BATCH + bvec

    def g_issue(k, h):
        pltpu.async_copy(x_hbm.at[gidxs[(k % 2) * NQ + h]],
                         bufs[k % 2].at[pl.ds(h * QR, QR)],
                         gsems[(k % 2) * NQ + h])

    def g_wait(k, h):
        pltpu.make_async_copy(x_hbm.at[gidxs[(k % 2) * NQ + h]],
                              bufs[k % 2].at[pl.ds(h * QR, QR)],
                              gsems[(k % 2) * NQ + h]).wait()

    def s_issue(k):
        pltpu.async_copy(bufs[k % 2], out_hbm.at[pl.ds((t0 + k) * BATCH,
                                                       BATCH)], ssems[k % 2])

    def s_wait(k):
        pltpu.make_async_copy(bufs[k % 2],
                              out_hbm.at[pl.ds((t0 + k) * BATCH, BATCH)],
                              ssems[k % 2]).wait()

    for k in (0, 1):
        for h in range(NQ):
            build_gidx(k, h)
            g_issue(k, h)
    for k in range(SPT):
        for h in range(NQ):
            g_wait(k, h)
        s_issue(k)
        if k + 2 < SPT:
            s_wait(k)
            for h in range(NQ):
                build_gidx(k + 2, h)
                g_issue(k + 2, h)
    s_wait(SPT - 2)
    s_wait(SPT - 1)

    @pl.when(wid < 8)
    def _():
        pltpu.sync_copy(pmT_hbm.at[pl.ds(KEEP, 8)], blk_v.at[pl.ds(0, 8)])
        bvec = wid * 8 + lanes
        s = plsc.load_gather(blk_v, [jnp.zeros((L,), jnp.int32), bvec],
                             mask=lanes < jnp.full((L,), 8, jnp.int32))
        g0[pl.ds(0, L)] = s * BATCH + bvec
        src = x_hbm.at[g0.at[pl.ds(0, 8)]]
        pltpu.async_copy(src, b0.at[pl.ds(0, 8)], gs0)
        pltpu.make_async_copy(src, b0.at[pl.ds(0, 8)], gs0).wait()
        dst = out_hbm.at[pl.ds(KEEP * BATCH + wid * 8, 8)]
        pltpu.async_copy(b0.at[pl.ds(0, 8)], dst, ss0)
        pltpu.make_async_copy(b0.at[pl.ds(0, 8)], dst, ss0).wait()


_mesh = plsc.VectorSubcoreMesh(
    core_axis_name="c", subcore_axis_name="s", num_cores=NC, num_subcores=NS
)


@jax.jit
def _run(x, noise):
    sel = pl.kernel(
        _sel_body,
        out_type=jax.ShapeDtypeStruct((BATCH * PMS,), jnp.int32),
        mesh=_mesh,
        scratch_types=[
            pltpu.VMEM((PATCH,), jnp.float32),
            pltpu.VMEM((OUT + 7,), jnp.int32),
        ],
        compiler_params=pltpu.CompilerParams(needs_layout_passes=False),
    )
    pm = sel(noise.reshape(BATCH * PATCH))
    pmT = pm.reshape(BATCH, PMS).T

    gat = pl.kernel(
        _gat_body,
        out_type=jax.ShapeDtypeStruct((OUT * BATCH, DIM), jnp.float32),
        mesh=_mesh,
        scratch_types=(
            [pltpu.VMEM((L, BATCH), jnp.int32)]
            + [pltpu.VMEM((L,), jnp.int32)] * 8
            + [pltpu.VMEM((BATCH, DIM), jnp.float32)] * 2
            + [pltpu.SemaphoreType.DMA] * 10
        ),
        compiler_params=pltpu.CompilerParams(needs_layout_passes=False),
    )
    x2d = x.transpose(1, 0, 2).reshape(SEQ * BATCH, DIM)
    out2d = gat(x2d, pmT)
    return out2d.reshape(OUT, BATCH, DIM).transpose(1, 0, 2)


def kernel(x, force_drop, noise):
    del force_drop
    return _run(x, noise)
